# baseline (speedup 1.0000x reference)
import numpy as np

import jax
import jax.numpy as jnp
from jax import lax
from jax.experimental import pallas as pl
from jax.experimental.pallas import tpu as pltpu
from jax.experimental.pallas import tpu_sc as plsc

H = 256
M = H * H
T = 50

NS = 16
CHUNK = M // NS
GRP = CHUNK // 16
NROW = CHUNK // 128
TRASH = M + 32
LIST_PAD = 128

_rt = ((np.arange(H, dtype=np.float32) - np.float32(128.0))
       * np.float32(0.8))
_rt = (_rt / np.float32(0.8) + np.float32(128.0)).astype(np.int32)
_RMAP_NP = (_rt[:, None] * H + _rt[None, :]).astype(np.int32)


def _k1_body(added_ref, orig_ref, boxes_ref, ego_ref,
             rankpack_ref, bounds_ref, bits_ref):
    f32 = jnp.float32
    ii = lax.broadcasted_iota(jnp.int32, (H, H), 0).astype(f32)
    jj = lax.broadcasted_iota(jnp.int32, (H, H), 1).astype(f32)
    su = (ii < jj).astype(f32)
    sl = (ii > jj).astype(f32)
    ones = jnp.ones((H, H), f32)

    def ranks(nzf):
        row_tot = jnp.dot(nzf, ones, preferred_element_type=f32)
        row_prefix = jnp.dot(sl, row_tot, preferred_element_type=f32)
        row_pre = jnp.dot(nzf, su, preferred_element_type=f32)
        return row_prefix + row_pre

    for c, b in ((0, 0), (0, 1), (1, 0), (1, 1)):
        if c == 0:
            s = (orig_ref[b, 1] + orig_ref[b, 2] + orig_ref[b, 3]
                 + orig_ref[b, 4] + orig_ref[b, 5])
        else:
            s = (added_ref[b, 0] + added_ref[b, 1] + added_ref[b, 2]
                 + added_ref[b, 3] + added_ref[b, 4])
        nz = (s != 0.0)
        nzf = nz.astype(f32)
        rank = ranks(nzf).astype(jnp.int32)
        rankpack_ref[c, b] = rank | (nz.astype(jnp.int32) << 30)

        role = b
        off = role * 32
        bounds_ref[c, off] = 0
        for sb in range(1, NS):
            cnt = jnp.sum((rank < sb * CHUNK).astype(jnp.int32))
            bounds_ref[c, off + sb] = cnt
        bounds_ref[c, off + NS] = M
        if role == 0:
            nzi = nz.astype(jnp.int32)
            bounds_ref[c, 64] = 0
            for sb in range(1, NS):
                cnt = jnp.sum(nzi * (ii < float(sb * NS)).astype(jnp.int32))
                bounds_ref[c, 64 + sb] = cnt
            bounds_ref[c, 64 + NS] = jnp.sum(nzi)
        else:
            bounds_ref[c, 96] = jnp.sum(nz.astype(jnp.int32))

    px_full = (ii - 128.0) * 0.8
    py_full = (jj - 128.0) * 0.8
    ego_x = ego_ref[1, 0]
    ego_y = ego_ref[1, 1]
    anyin = jnp.zeros((H, H), jnp.int32)
    for t in range(T):
        cx = boxes_ref[1, t, 0] - ego_x
        cy = boxes_ref[1, t, 1] - ego_y
        cz = boxes_ref[1, t, 2]
        dx = boxes_ref[1, t, 3]
        dy = boxes_ref[1, t, 4]
        dz = boxes_ref[1, t, 5]
        ry = boxes_ref[1, t, 6]
        cth = jnp.cos(-ry)
        sth = jnp.sin(-ry)
        px = px_full - cx
        py = py_full - cy
        lx = px * cth - py * sth
        ly = px * sth + py * cth
        pz = 0.0 - cz
        zok = jnp.logical_and(pz >= 0.0, pz <= dz)
        inb = (jnp.abs(lx) <= dx * 0.5) & (jnp.abs(ly) <= dy * 0.5) & zok
        anyin = anyin | inb.astype(jnp.int32)

    cc = lax.broadcasted_iota(jnp.int32, (H, NS), 0)
    ww = lax.broadcasted_iota(jnp.int32, (H, NS), 1)
    pf = jnp.where((cc >> 4) == ww, 1 << (cc & 15), 0).astype(f32)
    bits_ref[...] = jnp.dot(anyin.astype(f32), pf,
                            preferred_element_type=f32).astype(jnp.int32)


@jax.jit
def _k1(added_points, original_points, boxes, ego_loc):
    return pl.pallas_call(
        _k1_body,
        in_specs=[
            pl.BlockSpec(memory_space=pltpu.VMEM),
            pl.BlockSpec(memory_space=pltpu.VMEM),
            pl.BlockSpec(memory_space=pltpu.SMEM),
            pl.BlockSpec(memory_space=pltpu.SMEM),
        ],
        out_specs=[
            pl.BlockSpec(memory_space=pltpu.VMEM),
            pl.BlockSpec(memory_space=pltpu.SMEM),
            pl.BlockSpec(memory_space=pltpu.VMEM),
        ],
        out_shape=[
            jax.ShapeDtypeStruct((2, 2, H, H), jnp.int32),
            jax.ShapeDtypeStruct((2, 128), jnp.int32),
            jax.ShapeDtypeStruct((H, NS), jnp.int32),
        ],
    )(added_points, original_points, boxes, ego_loc)


WIN = 4096
TCW = CHUNK + 512


def _k2_body(rankpack_hbm, bounds_hbm, bits_hbm, out_hbm,
             lists_hbm, bitsb, win_a, win_b, chunk_a, chunk_b, bvec,
             sem_a, sem_b, sem_bits):
    c = lax.axis_index("c")
    s = lax.axis_index("s")
    lo_out = s * CHUNK
    lane = lax.iota(jnp.int32, 16)
    ones16 = jnp.ones((16,), jnp.int32)
    zeros16 = jnp.zeros((16,), jnp.int32)

    pltpu.sync_copy(bounds_hbm.at[pl.ds(c * 128, 128)], bvec)
    cp_bits = pltpu.async_copy(bits_hbm, bitsb, sem_bits)

    def rt_fix(k):
        return ((k % 5 == 4) & (k < 45)).astype(jnp.int32)

    def extract(off, k):
        va = bvec[pl.ds(off, 16)]
        vb = bvec[pl.ds(off + 16, 16)]
        return (jnp.sum(jnp.where(lane == k, va, 0))
                + jnp.sum(jnp.where(lane == (k - 16), vb, 0)))

    def zero_buf(buf):
        def z(i, carry):
            buf[pl.ds(i * 16, 16)] = zeros16
            return carry
        lax.fori_loop(0, GRP, z, 0, unroll=16)

    def win_bounds(vlo, vhi):
        wstart0 = (vlo >> 7) * 128
        wend = ((vhi + 127) >> 7) * 128
        nwin = (wend - wstart0 + WIN - 1) >> 12
        return wstart0, nwin

    def grp_bounds(vlo, vhi, wstart):
        glo = jnp.maximum(0, (vlo - wstart) >> 4)
        ghi = jnp.minimum(WIN // 16,
                          jnp.maximum(glo, (vhi - wstart + 15) >> 4))
        return glo, ghi

    writes = []
    for role, chunk, wsem in ((0, chunk_a, sem_a), (1, chunk_b, sem_b)):
        off = role * 32
        flo = extract(off, s)
        fhi = extract(off, s + 1)
        zero_buf(chunk)
        abase = (c * 2 + role) * M
        wstart0, nwin = win_bounds(flo, fhi)

        def wloop(w, carry, wstart0=wstart0, flo=flo, fhi=fhi,
                  abase=abase, chunk=chunk):
            wstart = pl.multiple_of(jnp.minimum(wstart0 + w * WIN, M - WIN),
                                    128)
            pltpu.sync_copy(rankpack_hbm.at[pl.ds(abase + wstart, WIN)],
                            win_a)
            glo, ghi = grp_bounds(flo, fhi, wstart)

            def g(i4, carry2):
                for u in range(4):
                    i = i4 * 4 + u
                    rp = win_a[pl.ds(i * 16, 16)]
                    rank = rp & 0xFFFF
                    nz = rp >> 30
                    f = wstart + i * 16 + lane
                    loc = rank - lo_out
                    m = (nz == 1) & (loc >= 0) & (loc < CHUNK)
                    plsc.store_scatter(chunk, [jnp.where(m, loc, 0)], f,
                                       mask=m)
                return carry2
            lax.fori_loop(glo >> 2, (ghi + 3) >> 2, g, 0)
            return carry
        lax.fori_loop(0, nwin, wloop, 0)
        writes.append(pltpu.async_copy(
            chunk, lists_hbm.at[pl.ds(abase + lo_out, CHUNK)], wsem))
    for wh in writes:
        wh.wait()
    plsc.subcore_barrier()

    cp_bits.wait()
    zero_buf(chunk_a)

    jlo = extract(64, s)
    jhi = jnp.minimum(M, extract(64, s + 1) + 272)
    wstart0, nwin = win_bounds(jlo, jhi)

    def wloop2(w, carry):
        wstart = pl.multiple_of(jnp.minimum(wstart0 + w * WIN, M - WIN), 128)
        ca = pltpu.async_copy(lists_hbm.at[pl.ds(c * 2 * M + wstart, WIN)],
                              win_a, sem_a)
        cb = pltpu.async_copy(
            lists_hbm.at[pl.ds((c * 2 + 1) * M + wstart, WIN)], win_b, sem_b)
        ca.wait()
        cb.wait()
        glo, ghi = grp_bounds(jlo, jhi, wstart)

        def g(i4, carry2):
            for u in range(4):
                i = i4 * 4 + u
                l0 = win_a[pl.ds(i * 16, 16)]
                l1 = win_b[pl.ds(i * 16, 16)]
                hx = l0 >> 8
                hy = l0 & 255
                tgt = ((hx - rt_fix(hx)) << 8) + hy - rt_fix(hy)
                wv = plsc.load_gather(bitsb, [l1 >> 4])
                v = (wv >> (l1 & 15)) & 1
                loc = tgt - lo_out
                m = (v == 1) & (loc >= 0) & (loc < CHUNK)
                plsc.store_scatter(chunk_a, [jnp.where(m, loc, 0)], ones16,
                                   mask=m)
            return carry2
        lax.fori_loop(glo >> 2, (ghi + 3) >> 2, g, 0)
        return carry
    lax.fori_loop(0, nwin, wloop2, 0)

    count0 = extract(64, 16)
    count1 = extract(96, 0)

    @pl.when(jnp.logical_and(s == 0, count0 < M))
    def _():
        te = jnp.minimum(M, jnp.maximum(count0, count1) + 1)
        tw0, tnwin = win_bounds(count0, te)

        def wloop3(w, carry):
            wstart = pl.multiple_of(jnp.minimum(tw0 + w * WIN, M - WIN), 128)
            pltpu.sync_copy(
                lists_hbm.at[pl.ds((c * 2 + 1) * M + wstart, WIN)], win_b)
            glo, ghi = grp_bounds(count0, te, wstart)

            def g(i, carry2):
                j = wstart + i * 16 + lane
                l1 = win_b[pl.ds(i * 16, 16)]
                wv = plsc.load_gather(bitsb, [l1 >> 4])
                v = (wv >> (l1 & 15)) & 1
                m = (j >= count0) & (v == 1)
                plsc.store_scatter(chunk_a, [zeros16], ones16, mask=m)
                return carry2
            lax.fori_loop(glo, ghi, g, 0)
            return carry
        lax.fori_loop(0, tnwin, wloop3, 0)

    pltpu.sync_copy(chunk_a, out_hbm.at[pl.ds(c * M + lo_out, CHUNK)])


@jax.jit
def _k2(rankpack, bounds, bits):
    mesh = plsc.VectorSubcoreMesh(core_axis_name="c", subcore_axis_name="s")
    return pl.kernel(
        _k2_body,
        out_type=[
            jax.ShapeDtypeStruct((2 * M,), jnp.int32),
            jax.ShapeDtypeStruct((4 * M,), jnp.int32),
        ],
        mesh=mesh,
        compiler_params=pltpu.CompilerParams(needs_layout_passes=False),
        scratch_types=[
            pltpu.VMEM((M // 16,), jnp.int32),
            pltpu.VMEM((WIN,), jnp.int32),
            pltpu.VMEM((WIN,), jnp.int32),
            pltpu.VMEM((CHUNK,), jnp.int32),
            pltpu.VMEM((CHUNK,), jnp.int32),
            pltpu.VMEM((128,), jnp.int32),
            pltpu.SemaphoreType.DMA,
            pltpu.SemaphoreType.DMA,
            pltpu.SemaphoreType.DMA,
        ],
    )(rankpack, bounds, bits)


def _k3_body(grids_ref, iou_ref):
    o = grids_ref[0] > 0
    p = grids_ref[1] > 0
    inter = jnp.sum((o & p).astype(jnp.float32))
    union = jnp.sum((o | p).astype(jnp.float32))
    iou_ref[0, 0] = inter / union


@jax.jit
def _k3(grids):
    return pl.pallas_call(
        _k3_body,
        in_specs=[pl.BlockSpec(memory_space=pltpu.VMEM)],
        out_specs=pl.BlockSpec(memory_space=pltpu.SMEM),
        out_shape=jax.ShapeDtypeStruct((1, 1), jnp.float32),
    )(grids)


def kernel(added_points, original_points, boxes, ego_loc):
    rankpack, bounds, bits = _k1(added_points, original_points,
                                 boxes, ego_loc)
    grids, _ = _k2(rankpack.reshape(4 * M),
                   bounds.reshape(256), bits.reshape(M // 16))
    iou = _k3(grids.reshape(2, H, H))
    return iou[0, 0]


# speedup vs baseline: 1.1867x; 1.1867x over previous
import numpy as np

import jax
import jax.numpy as jnp
from jax import lax
from jax.experimental import pallas as pl
from jax.experimental.pallas import tpu as pltpu
from jax.experimental.pallas import tpu_sc as plsc

H = 256
M = H * H
T = 50

NS = 16
CHUNK = M // NS
GRP = CHUNK // 16
NROW = CHUNK // 128
TRASH = M + 32
LIST_PAD = 128

_rt = ((np.arange(H, dtype=np.float32) - np.float32(128.0))
       * np.float32(0.8))
_rt = (_rt / np.float32(0.8) + np.float32(128.0)).astype(np.int32)
_RMAP_NP = (_rt[:, None] * H + _rt[None, :]).astype(np.int32)


def _k1_body(added_ref, orig_ref, boxes_ref, ego_ref,
             rankpack_ref, bounds_ref, bits_ref):
    f32 = jnp.float32
    ii = lax.broadcasted_iota(jnp.int32, (H, H), 0).astype(f32)
    jj = lax.broadcasted_iota(jnp.int32, (H, H), 1).astype(f32)
    su = (ii < jj).astype(f32)
    sl = (ii > jj).astype(f32)
    ones = jnp.ones((H, H), f32)

    def ranks(nzf):
        row_tot = jnp.dot(nzf, ones, preferred_element_type=f32)
        row_prefix = jnp.dot(sl, row_tot, preferred_element_type=f32)
        row_pre = jnp.dot(nzf, su, preferred_element_type=f32)
        return row_prefix + row_pre

    for c, b in ((0, 0), (0, 1), (1, 0), (1, 1)):
        if c == 0:
            s = (orig_ref[b, 1] + orig_ref[b, 2] + orig_ref[b, 3]
                 + orig_ref[b, 4] + orig_ref[b, 5])
        else:
            s = (added_ref[b, 0] + added_ref[b, 1] + added_ref[b, 2]
                 + added_ref[b, 3] + added_ref[b, 4])
        nz = (s != 0.0)
        nzf = nz.astype(f32)
        rank = ranks(nzf).astype(jnp.int32)
        rankpack_ref[c, b] = rank | (nz.astype(jnp.int32) << 30)

        role = b
        off = role * 32
        bounds_ref[c, off] = 0
        for sb in range(1, NS):
            cnt = jnp.sum((rank < sb * CHUNK).astype(jnp.int32))
            bounds_ref[c, off + sb] = cnt
        bounds_ref[c, off + NS] = M
        if role == 0:
            nzi = nz.astype(jnp.int32)
            bounds_ref[c, 64] = 0
            for sb in range(1, NS):
                cnt = jnp.sum(nzi * (ii < float(sb * NS)).astype(jnp.int32))
                bounds_ref[c, 64 + sb] = cnt
            bounds_ref[c, 64 + NS] = jnp.sum(nzi)
        else:
            bounds_ref[c, 96] = jnp.sum(nz.astype(jnp.int32))

    px_col = (lax.broadcasted_iota(jnp.int32, (H, 1), 0).astype(f32)
              - 128.0) * 0.8
    py_row = (lax.broadcasted_iota(jnp.int32, (1, H), 1).astype(f32)
              - 128.0) * 0.8
    ego_x = ego_ref[1, 0]
    ego_y = ego_ref[1, 1]
    anyin = jnp.zeros((H, H), jnp.int32)
    for t in range(T):
        cx = boxes_ref[1, t, 0] - ego_x
        cy = boxes_ref[1, t, 1] - ego_y
        cz = boxes_ref[1, t, 2]
        dx = boxes_ref[1, t, 3]
        dy = boxes_ref[1, t, 4]
        dz = boxes_ref[1, t, 5]
        ry = boxes_ref[1, t, 6]
        cth = jnp.cos(-ry)
        sth = jnp.sin(-ry)
        ax = px_col - cx
        by = py_row - cy
        axc = ax * cth
        axs = ax * sth
        byc = by * cth
        bys = by * sth
        lx = axc - bys
        ly = axs + byc
        pz = 0.0 - cz
        zok = jnp.logical_and(pz >= 0.0, pz <= dz)
        inb = (jnp.abs(lx) <= dx * 0.5) & (jnp.abs(ly) <= dy * 0.5) & zok
        anyin = anyin | inb.astype(jnp.int32)

    cc = lax.broadcasted_iota(jnp.int32, (H, NS), 0)
    ww = lax.broadcasted_iota(jnp.int32, (H, NS), 1)
    pf = jnp.where((cc >> 4) == ww, 1 << (cc & 15), 0).astype(f32)
    bits_ref[...] = jnp.dot(anyin.astype(f32), pf,
                            preferred_element_type=f32).astype(jnp.int32)


@jax.jit
def _k1(added_points, original_points, boxes, ego_loc):
    return pl.pallas_call(
        _k1_body,
        in_specs=[
            pl.BlockSpec(memory_space=pltpu.VMEM),
            pl.BlockSpec(memory_space=pltpu.VMEM),
            pl.BlockSpec(memory_space=pltpu.SMEM),
            pl.BlockSpec(memory_space=pltpu.SMEM),
        ],
        out_specs=[
            pl.BlockSpec(memory_space=pltpu.VMEM),
            pl.BlockSpec(memory_space=pltpu.SMEM),
            pl.BlockSpec(memory_space=pltpu.VMEM),
        ],
        out_shape=[
            jax.ShapeDtypeStruct((2, 2, H, H), jnp.int32),
            jax.ShapeDtypeStruct((2, 128), jnp.int32),
            jax.ShapeDtypeStruct((H, NS), jnp.int32),
        ],
    )(added_points, original_points, boxes, ego_loc)


WIN = 4096
TCW = CHUNK + 512


def _k2_body(rankpack_hbm, bounds_hbm, bits_hbm, out_hbm,
             lists_hbm, bitsb, win_a, win_b, chunk_a, chunk_b, bvec,
             sem_a, sem_b, sem_bits):
    c = lax.axis_index("c")
    s = lax.axis_index("s")
    lo_out = s * CHUNK
    lane = lax.iota(jnp.int32, 16)
    ones16 = jnp.ones((16,), jnp.int32)
    zeros16 = jnp.zeros((16,), jnp.int32)

    pltpu.sync_copy(bounds_hbm.at[pl.ds(c * 128, 128)], bvec)
    cp_bits = pltpu.async_copy(bits_hbm, bitsb, sem_bits)

    def rt_fix(k):
        return ((k % 5 == 4) & (k < 45)).astype(jnp.int32)

    def extract(off, k):
        va = bvec[pl.ds(off, 16)]
        vb = bvec[pl.ds(off + 16, 16)]
        return (jnp.sum(jnp.where(lane == k, va, 0))
                + jnp.sum(jnp.where(lane == (k - 16), vb, 0)))

    def zero_buf(buf):
        def z(i, carry):
            buf[pl.ds(i * 16, 16)] = zeros16
            return carry
        lax.fori_loop(0, GRP, z, 0, unroll=16)

    def win_bounds(vlo, vhi):
        wstart0 = (vlo >> 7) * 128
        wend = ((vhi + 127) >> 7) * 128
        nwin = (wend - wstart0 + WIN - 1) >> 12
        return wstart0, nwin

    def grp_bounds(vlo, vhi, wstart):
        glo = jnp.maximum(0, (vlo - wstart) >> 4)
        ghi = jnp.minimum(WIN // 16,
                          jnp.maximum(glo, (vhi - wstart + 15) >> 4))
        return glo, ghi

    writes = []
    for role, chunk, wsem in ((0, chunk_a, sem_a), (1, chunk_b, sem_b)):
        off = role * 32
        flo = extract(off, s)
        fhi = extract(off, s + 1)
        zero_buf(chunk)
        abase = (c * 2 + role) * M
        wstart0, nwin = win_bounds(flo, fhi)

        def wloop(w, carry, wstart0=wstart0, flo=flo, fhi=fhi,
                  abase=abase, chunk=chunk):
            wstart = pl.multiple_of(jnp.minimum(wstart0 + w * WIN, M - WIN),
                                    128)
            pltpu.sync_copy(rankpack_hbm.at[pl.ds(abase + wstart, WIN)],
                            win_a)
            glo, ghi = grp_bounds(flo, fhi, wstart)

            def g(i4, carry2):
                for u in range(4):
                    i = i4 * 4 + u
                    rp = win_a[pl.ds(i * 16, 16)]
                    rank = rp & 0xFFFF
                    nz = rp >> 30
                    f = wstart + i * 16 + lane
                    loc = rank - lo_out
                    m = (nz == 1) & (loc >= 0) & (loc < CHUNK)
                    plsc.store_scatter(chunk, [jnp.where(m, loc, 0)], f,
                                       mask=m)
                return carry2
            lax.fori_loop(glo >> 2, (ghi + 3) >> 2, g, 0)
            return carry
        lax.fori_loop(0, nwin, wloop, 0)
        writes.append(pltpu.async_copy(
            chunk, lists_hbm.at[pl.ds(abase + lo_out, CHUNK)], wsem))
    for wh in writes:
        wh.wait()
    plsc.subcore_barrier()

    cp_bits.wait()
    zero_buf(chunk_a)

    jlo = extract(64, s)
    jhi = jnp.minimum(M, extract(64, s + 1) + 272)
    wstart0, nwin = win_bounds(jlo, jhi)

    def wloop2(w, carry):
        wstart = pl.multiple_of(jnp.minimum(wstart0 + w * WIN, M - WIN), 128)
        ca = pltpu.async_copy(lists_hbm.at[pl.ds(c * 2 * M + wstart, WIN)],
                              win_a, sem_a)
        cb = pltpu.async_copy(
            lists_hbm.at[pl.ds((c * 2 + 1) * M + wstart, WIN)], win_b, sem_b)
        ca.wait()
        cb.wait()
        glo, ghi = grp_bounds(jlo, jhi, wstart)

        def g(i4, carry2):
            for u in range(4):
                i = i4 * 4 + u
                l0 = win_a[pl.ds(i * 16, 16)]
                l1 = win_b[pl.ds(i * 16, 16)]
                hx = l0 >> 8
                hy = l0 & 255
                tgt = ((hx - rt_fix(hx)) << 8) + hy - rt_fix(hy)
                wv = plsc.load_gather(bitsb, [l1 >> 4])
                v = (wv >> (l1 & 15)) & 1
                loc = tgt - lo_out
                m = (v == 1) & (loc >= 0) & (loc < CHUNK)
                plsc.store_scatter(chunk_a, [jnp.where(m, loc, 0)], ones16,
                                   mask=m)
            return carry2
        lax.fori_loop(glo >> 2, (ghi + 3) >> 2, g, 0)
        return carry
    lax.fori_loop(0, nwin, wloop2, 0)

    count0 = extract(64, 16)
    count1 = extract(96, 0)

    @pl.when(jnp.logical_and(s == 0, count0 < M))
    def _():
        te = jnp.minimum(M, jnp.maximum(count0, count1) + 1)
        tw0, tnwin = win_bounds(count0, te)

        def wloop3(w, carry):
            wstart = pl.multiple_of(jnp.minimum(tw0 + w * WIN, M - WIN), 128)
            pltpu.sync_copy(
                lists_hbm.at[pl.ds((c * 2 + 1) * M + wstart, WIN)], win_b)
            glo, ghi = grp_bounds(count0, te, wstart)

            def g(i, carry2):
                j = wstart + i * 16 + lane
                l1 = win_b[pl.ds(i * 16, 16)]
                wv = plsc.load_gather(bitsb, [l1 >> 4])
                v = (wv >> (l1 & 15)) & 1
                m = (j >= count0) & (v == 1)
                plsc.store_scatter(chunk_a, [zeros16], ones16, mask=m)
                return carry2
            lax.fori_loop(glo, ghi, g, 0)
            return carry
        lax.fori_loop(0, tnwin, wloop3, 0)

    pltpu.sync_copy(chunk_a, out_hbm.at[pl.ds(c * M + lo_out, CHUNK)])


@jax.jit
def _k2(rankpack, bounds, bits):
    mesh = plsc.VectorSubcoreMesh(core_axis_name="c", subcore_axis_name="s")
    return pl.kernel(
        _k2_body,
        out_type=[
            jax.ShapeDtypeStruct((2 * M,), jnp.int32),
            jax.ShapeDtypeStruct((4 * M,), jnp.int32),
        ],
        mesh=mesh,
        compiler_params=pltpu.CompilerParams(needs_layout_passes=False),
        scratch_types=[
            pltpu.VMEM((M // 16,), jnp.int32),
            pltpu.VMEM((WIN,), jnp.int32),
            pltpu.VMEM((WIN,), jnp.int32),
            pltpu.VMEM((CHUNK,), jnp.int32),
            pltpu.VMEM((CHUNK,), jnp.int32),
            pltpu.VMEM((128,), jnp.int32),
            pltpu.SemaphoreType.DMA,
            pltpu.SemaphoreType.DMA,
            pltpu.SemaphoreType.DMA,
        ],
    )(rankpack, bounds, bits)


def _k3_body(grids_ref, iou_ref):
    o = grids_ref[0] > 0
    p = grids_ref[1] > 0
    inter = jnp.sum((o & p).astype(jnp.float32))
    union = jnp.sum((o | p).astype(jnp.float32))
    iou_ref[0, 0] = inter / union


@jax.jit
def _k3(grids):
    return pl.pallas_call(
        _k3_body,
        in_specs=[pl.BlockSpec(memory_space=pltpu.VMEM)],
        out_specs=pl.BlockSpec(memory_space=pltpu.SMEM),
        out_shape=jax.ShapeDtypeStruct((1, 1), jnp.float32),
    )(grids)


def kernel(added_points, original_points, boxes, ego_loc):
    rankpack, bounds, bits = _k1(added_points, original_points,
                                 boxes, ego_loc)
    grids, _ = _k2(rankpack.reshape(4 * M),
                   bounds.reshape(256), bits.reshape(M // 16))
    iou = _k3(grids.reshape(2, H, H))
    return iou[0, 0]


# speedup vs baseline: 1.2697x; 1.0699x over previous
import numpy as np

import jax
import jax.numpy as jnp
from jax import lax
from jax.experimental import pallas as pl
from jax.experimental.pallas import tpu as pltpu
from jax.experimental.pallas import tpu_sc as plsc

H = 256
M = H * H
T = 50

NS = 16
CHUNK = M // NS
GRP = CHUNK // 16
NROW = CHUNK // 128
TRASH = M + 32
LIST_PAD = 128

_rt = ((np.arange(H, dtype=np.float32) - np.float32(128.0))
       * np.float32(0.8))
_rt = (_rt / np.float32(0.8) + np.float32(128.0)).astype(np.int32)
_RMAP_NP = (_rt[:, None] * H + _rt[None, :]).astype(np.int32)


def _k1_body(added_ref, orig_ref, boxes_ref, ego_ref,
             rankpack_ref, bounds_ref, bits_ref):
    f32 = jnp.float32
    ii = lax.broadcasted_iota(jnp.int32, (H, H), 0).astype(f32)
    jj = lax.broadcasted_iota(jnp.int32, (H, H), 1).astype(f32)
    su = (ii < jj).astype(f32)
    sl = (ii > jj).astype(f32)
    ones = jnp.ones((H, H), f32)

    def ranks(nzf):
        row_tot = jnp.dot(nzf, ones, preferred_element_type=f32)
        row_prefix = jnp.dot(sl, row_tot, preferred_element_type=f32)
        row_pre = jnp.dot(nzf, su, preferred_element_type=f32)
        return row_prefix + row_pre

    for c, b in ((0, 0), (0, 1), (1, 0), (1, 1)):
        if c == 0:
            s = (orig_ref[b, 1] + orig_ref[b, 2] + orig_ref[b, 3]
                 + orig_ref[b, 4] + orig_ref[b, 5])
        else:
            s = (added_ref[b, 0] + added_ref[b, 1] + added_ref[b, 2]
                 + added_ref[b, 3] + added_ref[b, 4])
        nz = (s != 0.0)
        nzf = nz.astype(f32)
        rank = ranks(nzf).astype(jnp.int32)
        rankpack_ref[c, b] = rank | (nz.astype(jnp.int32) << 30)

        role = b
        off = role * 32
        bounds_ref[c, off] = 0
        for sb in range(1, NS):
            cnt = jnp.sum((rank < sb * CHUNK).astype(jnp.int32))
            bounds_ref[c, off + sb] = cnt
        bounds_ref[c, off + NS] = M
        if role == 0:
            nzi = nz.astype(jnp.int32)
            bounds_ref[c, 64] = 0
            for sb in range(1, NS):
                cnt = jnp.sum(nzi * (ii < float(sb * NS)).astype(jnp.int32))
                bounds_ref[c, 64 + sb] = cnt
            bounds_ref[c, 64 + NS] = jnp.sum(nzi)
        else:
            bounds_ref[c, 96] = jnp.sum(nz.astype(jnp.int32))

    px_col = (lax.broadcasted_iota(jnp.int32, (H, 1), 0).astype(f32)
              - 128.0) * 0.8
    py_row = (lax.broadcasted_iota(jnp.int32, (1, H), 1).astype(f32)
              - 128.0) * 0.8
    ego_x = ego_ref[1, 0]
    ego_y = ego_ref[1, 1]
    anyin = jnp.zeros((H, H), jnp.int32)
    for t in range(T):
        cx = boxes_ref[1, t, 0] - ego_x
        cy = boxes_ref[1, t, 1] - ego_y
        cz = boxes_ref[1, t, 2]
        dx = boxes_ref[1, t, 3]
        dy = boxes_ref[1, t, 4]
        dz = boxes_ref[1, t, 5]
        ry = boxes_ref[1, t, 6]
        cth = jnp.cos(-ry)
        sth = jnp.sin(-ry)
        ax = px_col - cx
        by = py_row - cy
        axc = ax * cth
        axs = ax * sth
        byc = by * cth
        bys = by * sth
        lx = axc - bys
        ly = axs + byc
        pz = 0.0 - cz
        zok = jnp.logical_and(pz >= 0.0, pz <= dz)
        inb = (jnp.abs(lx) <= dx * 0.5) & (jnp.abs(ly) <= dy * 0.5) & zok
        anyin = anyin | inb.astype(jnp.int32)

    ww = lax.broadcasted_iota(jnp.int32, (NS, H), 0)
    cc = lax.broadcasted_iota(jnp.int32, (NS, H), 1)
    pf = jnp.where((cc >> 4) == ww, 1 << (cc & 15), 0).astype(f32)
    bits_ref[...] = lax.dot_general(
        pf, anyin.astype(f32), (((1,), (1,)), ((), ())),
        preferred_element_type=f32).astype(jnp.int32)


@jax.jit
def _k1(added_points, original_points, boxes, ego_loc):
    return pl.pallas_call(
        _k1_body,
        in_specs=[
            pl.BlockSpec(memory_space=pltpu.VMEM),
            pl.BlockSpec(memory_space=pltpu.VMEM),
            pl.BlockSpec(memory_space=pltpu.SMEM),
            pl.BlockSpec(memory_space=pltpu.SMEM),
        ],
        out_specs=[
            pl.BlockSpec(memory_space=pltpu.VMEM),
            pl.BlockSpec(memory_space=pltpu.SMEM),
            pl.BlockSpec(memory_space=pltpu.VMEM),
        ],
        out_shape=[
            jax.ShapeDtypeStruct((2, 2, H, H), jnp.int32),
            jax.ShapeDtypeStruct((2, 128), jnp.int32),
            jax.ShapeDtypeStruct((NS, H), jnp.int32),
        ],
    )(added_points, original_points, boxes, ego_loc)


WIN = 4096
WROW = WIN // 256


def _k2_body(rankpack_hbm, bounds_hbm, bits_hbm, out_hbm, lists_hbm,
             bitsb, win_a, win_b, chunk_a, chunk_b, bvec,
             sem_a, sem_b, sem_bits):
    c = lax.axis_index("c")
    s = lax.axis_index("s")
    lo_out = s * CHUNK
    orow = pl.multiple_of(s * (CHUNK // 256), 8)
    lane = lax.iota(jnp.int32, 16)
    ones16 = jnp.ones((16,), jnp.int32)
    zeros16 = jnp.zeros((16,), jnp.int32)

    pltpu.sync_copy(bounds_hbm, bvec)
    cp_bits = pltpu.async_copy(bits_hbm, bitsb, sem_bits)

    def rt_fix(k):
        q = (k * 52429) >> 18
        return ((k - q * 5 == 4) & (k < 45)).astype(jnp.int32)

    def extract(off, k):
        va = bvec[c, pl.ds(off, 16)]
        vb = bvec[c, pl.ds(off + 16, 16)]
        return (jnp.sum(jnp.where(lane == k, va, 0))
                + jnp.sum(jnp.where(lane == (k - 16), vb, 0)))

    def zero_buf(buf):
        def z(i, carry):
            buf[i >> 4, pl.ds((i & 15) * 16, 16)] = zeros16
            return carry
        lax.fori_loop(0, GRP, z, 0, unroll=16)

    def win_bounds(vlo, vhi):
        wstart0 = (vlo >> 11) * 2048
        wend = ((vhi + 2047) >> 11) * 2048
        nwin = (wend - wstart0 + WIN - 1) >> 12
        return wstart0, nwin

    def grp_bounds(vlo, vhi, wstart):
        glo = jnp.maximum(0, (vlo - wstart) >> 4)
        ghi = jnp.minimum(WIN // 16,
                          jnp.maximum(glo, (vhi - wstart + 15) >> 4))
        return glo, ghi

    writes = []
    for role, chunk, wsem in ((0, chunk_a, sem_a), (1, chunk_b, sem_b)):
        off = role * 32
        flo = extract(off, s)
        fhi = extract(off, s + 1)
        zero_buf(chunk)
        wstart0, nwin = win_bounds(flo, fhi)

        def wloop(w, carry, wstart0=wstart0, flo=flo, fhi=fhi,
                  role=role, chunk=chunk):
            wstart = pl.multiple_of(jnp.minimum(wstart0 + w * WIN,
                                                M - WIN), 2048)
            wr = pl.multiple_of(wstart >> 8, 8)
            pltpu.sync_copy(
                rankpack_hbm.at[c, role, pl.ds(wr, WROW), :], win_a)
            glo, ghi = grp_bounds(flo, fhi, wstart)

            def g(i4, carry2):
                for u in range(4):
                    i = i4 * 4 + u
                    rp = win_a[i >> 4, pl.ds((i & 15) * 16, 16)]
                    rank = rp & 0xFFFF
                    nz = rp >> 30
                    f = wstart + i * 16 + lane
                    loc = rank - lo_out
                    m = (nz == 1) & (loc >= 0) & (loc < CHUNK)
                    lc = jnp.where(m, loc, 0)
                    plsc.store_scatter(chunk, [lc >> 8, lc & 255], f,
                                       mask=m)
                return carry2
            lax.fori_loop(glo >> 2, (ghi + 3) >> 2, g, 0)
            return carry
        lax.fori_loop(0, nwin, wloop, 0)
        writes.append(pltpu.async_copy(
            chunk, lists_hbm.at[c, role, pl.ds(orow, WROW), :], wsem))
    for wh in writes:
        wh.wait()
    plsc.subcore_barrier()

    cp_bits.wait()
    zero_buf(chunk_a)

    jlo = extract(64, s)
    jhi = jnp.minimum(M, extract(64, s + 1) + 272)
    wstart0, nwin = win_bounds(jlo, jhi)

    def wloop2(w, carry):
        wstart = pl.multiple_of(jnp.minimum(wstart0 + w * WIN, M - WIN),
                                2048)
        wr = pl.multiple_of(wstart >> 8, 8)
        ca = pltpu.async_copy(lists_hbm.at[c, 0, pl.ds(wr, WROW), :],
                              win_a, sem_a)
        cb = pltpu.async_copy(lists_hbm.at[c, 1, pl.ds(wr, WROW), :],
                              win_b, sem_b)
        ca.wait()
        cb.wait()
        glo, ghi = grp_bounds(jlo, jhi, wstart)

        def g(i4, carry2):
            for u in range(4):
                i = i4 * 4 + u
                l0 = win_a[i >> 4, pl.ds((i & 15) * 16, 16)]
                l1 = win_b[i >> 4, pl.ds((i & 15) * 16, 16)]
                hx = l0 >> 8
                hy = l0 & 255
                tgt = ((hx - rt_fix(hx)) << 8) + hy - rt_fix(hy)
                w1 = l1 >> 4
                wv = plsc.load_gather(bitsb, [w1 & 15, w1 >> 4])
                v = (wv >> (l1 & 15)) & 1
                loc = tgt - lo_out
                m = (v == 1) & (loc >= 0) & (loc < CHUNK)
                lc = jnp.where(m, loc, 0)
                plsc.store_scatter(chunk_a, [lc >> 8, lc & 255], ones16,
                                   mask=m)
            return carry2
        lax.fori_loop(glo >> 2, (ghi + 3) >> 2, g, 0)
        return carry
    lax.fori_loop(0, nwin, wloop2, 0)

    count0 = extract(64, 16)
    count1 = extract(96, 0)

    @pl.when(jnp.logical_and(s == 0, count0 < M))
    def _():
        te = jnp.minimum(M, jnp.maximum(count0, count1) + 1)
        tw0, tnwin = win_bounds(count0, te)

        def wloop3(w, carry):
            wstart = pl.multiple_of(jnp.minimum(tw0 + w * WIN, M - WIN),
                                    2048)
            wr = pl.multiple_of(wstart >> 8, 8)
            pltpu.sync_copy(lists_hbm.at[c, 1, pl.ds(wr, WROW), :], win_b)
            glo, ghi = grp_bounds(count0, te, wstart)

            def g(i, carry2):
                j = wstart + i * 16 + lane
                l1 = win_b[i >> 4, pl.ds((i & 15) * 16, 16)]
                w1 = l1 >> 4
                wv = plsc.load_gather(bitsb, [w1 & 15, w1 >> 4])
                v = (wv >> (l1 & 15)) & 1
                m = (j >= count0) & (v == 1)
                plsc.store_scatter(chunk_a, [zeros16, zeros16], ones16,
                                   mask=m)
                return carry2
            lax.fori_loop(glo, ghi, g, 0)
            return carry
        lax.fori_loop(0, tnwin, wloop3, 0)

    pltpu.sync_copy(chunk_a, out_hbm.at[c, pl.ds(orow, WROW), :])


@jax.jit
def _k2(rankpack, bounds, bits):
    mesh = plsc.VectorSubcoreMesh(core_axis_name="c", subcore_axis_name="s")
    return pl.kernel(
        _k2_body,
        out_type=[
            jax.ShapeDtypeStruct((2, H, H), jnp.int32),
            jax.ShapeDtypeStruct((2, 2, H, H), jnp.int32),
        ],
        mesh=mesh,
        compiler_params=pltpu.CompilerParams(needs_layout_passes=False,
                                             use_tc_tiling_on_sc=True),
        scratch_types=[
            pltpu.VMEM((NS, H), jnp.int32),
            pltpu.VMEM((WROW, H), jnp.int32),
            pltpu.VMEM((WROW, H), jnp.int32),
            pltpu.VMEM((CHUNK // 256, H), jnp.int32),
            pltpu.VMEM((CHUNK // 256, H), jnp.int32),
            pltpu.VMEM((2, 128), jnp.int32),
            pltpu.SemaphoreType.DMA,
            pltpu.SemaphoreType.DMA,
            pltpu.SemaphoreType.DMA,
        ],
    )(rankpack, bounds, bits)


def _k3_body(grids_ref, iou_ref):
    o = grids_ref[0] > 0
    p = grids_ref[1] > 0
    inter = jnp.sum((o & p).astype(jnp.float32))
    union = jnp.sum((o | p).astype(jnp.float32))
    iou_ref[0, 0] = inter / union


@jax.jit
def _k3(grids):
    return pl.pallas_call(
        _k3_body,
        in_specs=[pl.BlockSpec(memory_space=pltpu.VMEM)],
        out_specs=pl.BlockSpec(memory_space=pltpu.SMEM),
        out_shape=jax.ShapeDtypeStruct((1, 1), jnp.float32),
    )(grids)


def kernel(added_points, original_points, boxes, ego_loc):
    rankpack, bounds, bits = _k1(added_points, original_points,
                                 boxes, ego_loc)
    grids, _ = _k2(rankpack, bounds, bits)
    iou = _k3(grids)
    return iou[0, 0]


# speedup vs baseline: 1.2839x; 1.0112x over previous
import numpy as np

import jax
import jax.numpy as jnp
from jax import lax
from jax.experimental import pallas as pl
from jax.experimental.pallas import tpu as pltpu
from jax.experimental.pallas import tpu_sc as plsc

H = 256
M = H * H
T = 50

NS = 16
CHUNK = M // NS
GRP = CHUNK // 16
NROW = CHUNK // 128
TRASH = M + 32
LIST_PAD = 128

_rt = ((np.arange(H, dtype=np.float32) - np.float32(128.0))
       * np.float32(0.8))
_rt = (_rt / np.float32(0.8) + np.float32(128.0)).astype(np.int32)
_RMAP_NP = (_rt[:, None] * H + _rt[None, :]).astype(np.int32)


def _k1_body(added_ref, orig_ref, boxes_ref, ego_ref,
             rankpack_ref, bounds_ref, bits_ref):
    f32 = jnp.float32
    ii = lax.broadcasted_iota(jnp.int32, (H, H), 0).astype(f32)
    jj = lax.broadcasted_iota(jnp.int32, (H, H), 1).astype(f32)
    su = (ii < jj).astype(f32)
    sl = (ii > jj).astype(f32)
    ones = jnp.ones((H, H), f32)

    def ranks(nzf):
        row_tot = jnp.dot(nzf, ones, preferred_element_type=f32)
        row_prefix = jnp.dot(sl, row_tot, preferred_element_type=f32)
        row_pre = jnp.dot(nzf, su, preferred_element_type=f32)
        return row_prefix + row_pre

    for c, b in ((0, 0), (0, 1), (1, 0), (1, 1)):
        if c == 0:
            s = (orig_ref[b, 1] + orig_ref[b, 2] + orig_ref[b, 3]
                 + orig_ref[b, 4] + orig_ref[b, 5])
        else:
            s = (added_ref[b, 0] + added_ref[b, 1] + added_ref[b, 2]
                 + added_ref[b, 3] + added_ref[b, 4])
        nz = (s != 0.0)
        nzf = nz.astype(f32)
        rank = ranks(nzf).astype(jnp.int32)
        rankpack_ref[c, b] = rank | (nz.astype(jnp.int32) << 30)

        role = b
        off = role * 32
        bounds_ref[c, off] = 0
        for sb in range(1, NS):
            cnt = jnp.sum((rank < sb * CHUNK).astype(jnp.int32))
            bounds_ref[c, off + sb] = cnt
        bounds_ref[c, off + NS] = M
        if role == 0:
            nzi = nz.astype(jnp.int32)
            bounds_ref[c, 64] = 0
            for sb in range(1, NS):
                cnt = jnp.sum(nzi * (ii < float(sb * NS)).astype(jnp.int32))
                bounds_ref[c, 64 + sb] = cnt
            bounds_ref[c, 64 + NS] = jnp.sum(nzi)
        else:
            bounds_ref[c, 96] = jnp.sum(nz.astype(jnp.int32))

    px_col = (lax.broadcasted_iota(jnp.int32, (H, 1), 0).astype(f32)
              - 128.0) * 0.8
    py_row = (lax.broadcasted_iota(jnp.int32, (1, H), 1).astype(f32)
              - 128.0) * 0.8
    ego_x = ego_ref[1, 0]
    ego_y = ego_ref[1, 1]
    anyin = jnp.zeros((H, H), jnp.int32)
    for t in range(T):
        cx = boxes_ref[1, t, 0] - ego_x
        cy = boxes_ref[1, t, 1] - ego_y
        cz = boxes_ref[1, t, 2]
        dx = boxes_ref[1, t, 3]
        dy = boxes_ref[1, t, 4]
        dz = boxes_ref[1, t, 5]
        ry = boxes_ref[1, t, 6]
        cth = jnp.cos(-ry)
        sth = jnp.sin(-ry)
        ax = px_col - cx
        by = py_row - cy
        axc = ax * cth
        axs = ax * sth
        byc = by * cth
        bys = by * sth
        lx = axc - bys
        ly = axs + byc
        pz = 0.0 - cz
        zok = jnp.logical_and(pz >= 0.0, pz <= dz)
        inb = (jnp.abs(lx) <= dx * 0.5) & (jnp.abs(ly) <= dy * 0.5) & zok
        anyin = anyin | inb.astype(jnp.int32)

    ww = lax.broadcasted_iota(jnp.int32, (NS, H), 0)
    cc = lax.broadcasted_iota(jnp.int32, (NS, H), 1)
    pf = jnp.where((cc >> 4) == ww, 1 << (cc & 15), 0).astype(f32)
    bits_ref[...] = lax.dot_general(
        pf, anyin.astype(f32), (((1,), (1,)), ((), ())),
        preferred_element_type=f32).astype(jnp.int32)


@jax.jit
def _k1(added_points, original_points, boxes, ego_loc):
    return pl.pallas_call(
        _k1_body,
        in_specs=[
            pl.BlockSpec(memory_space=pltpu.VMEM),
            pl.BlockSpec(memory_space=pltpu.VMEM),
            pl.BlockSpec(memory_space=pltpu.SMEM),
            pl.BlockSpec(memory_space=pltpu.SMEM),
        ],
        out_specs=[
            pl.BlockSpec(memory_space=pltpu.VMEM),
            pl.BlockSpec(memory_space=pltpu.SMEM),
            pl.BlockSpec(memory_space=pltpu.VMEM),
        ],
        out_shape=[
            jax.ShapeDtypeStruct((2, 2, H, H), jnp.int32),
            jax.ShapeDtypeStruct((2, 128), jnp.int32),
            jax.ShapeDtypeStruct((NS, H), jnp.int32),
        ],
    )(added_points, original_points, boxes, ego_loc)


WIN = 8192
WROW = WIN // 256


def _k2_body(rankpack_hbm, bounds_hbm, bits_hbm, out_hbm, lists_hbm,
             bitsb, win_a, win_b, chunk_a, chunk_b, bvec,
             sem_a, sem_b, sem_bits):
    c = lax.axis_index("c")
    s = lax.axis_index("s")
    lo_out = s * CHUNK
    orow = pl.multiple_of(s * (CHUNK // 256), 8)
    lane = lax.iota(jnp.int32, 16)
    ones16 = jnp.ones((16,), jnp.int32)
    zeros16 = jnp.zeros((16,), jnp.int32)

    pltpu.sync_copy(bounds_hbm, bvec)
    cp_bits = pltpu.async_copy(bits_hbm, bitsb, sem_bits)

    def rt_fix(k):
        q = (k * 52429) >> 18
        return ((k - q * 5 == 4) & (k < 45)).astype(jnp.int32)

    def extract(off, k):
        va = bvec[c, pl.ds(off, 16)]
        vb = bvec[c, pl.ds(off + 16, 16)]
        return (jnp.sum(jnp.where(lane == k, va, 0))
                + jnp.sum(jnp.where(lane == (k - 16), vb, 0)))

    def zero_buf(buf):
        def z(i, carry):
            buf[i >> 4, pl.ds((i & 15) * 16, 16)] = zeros16
            return carry
        lax.fori_loop(0, GRP, z, 0, unroll=16)

    def win_bounds(vlo, vhi):
        wstart0 = (vlo >> 11) * 2048
        wend = ((vhi + 2047) >> 11) * 2048
        nwin = (wend - wstart0 + WIN - 1) >> 13
        return wstart0, nwin

    def grp_bounds(vlo, vhi, wstart):
        glo = jnp.maximum(0, (vlo - wstart) >> 4)
        ghi = jnp.minimum(WIN // 16,
                          jnp.maximum(glo, (vhi - wstart + 15) >> 4))
        return glo, ghi

    def wstart_of(ws0, w):
        return pl.multiple_of(jnp.minimum(ws0 + w * WIN, M - WIN), 2048)

    def rank_win(role, wstart, buf, sem):
        wr = pl.multiple_of(wstart >> 8, 8)
        return pltpu.async_copy(
            rankpack_hbm.at[c, role, pl.ds(wr, WROW), :], buf, sem)

    def p1_scan(buf, chunk, wstart, flo, fhi):
        glo, ghi = grp_bounds(flo, fhi, wstart)

        def g(i4, carry2):
            for u in range(4):
                i = i4 * 4 + u
                rp = buf[i >> 4, pl.ds((i & 15) * 16, 16)]
                rank = rp & 0xFFFF
                nz = rp >> 30
                f = wstart + i * 16 + lane
                loc = rank - lo_out
                m = (nz == 1) & (loc >= 0) & (loc < CHUNK)
                lc = jnp.where(m, loc, 0)
                plsc.store_scatter(chunk, [lc >> 8, lc & 255], f, mask=m)
            return carry2
        lax.fori_loop(glo >> 2, (ghi + 3) >> 2, g, 0)

    f0lo = extract(0, s)
    f0hi = extract(0, s + 1)
    f1lo = extract(32, s)
    f1hi = extract(32, s + 1)
    w0start0, n0 = win_bounds(f0lo, f0hi)
    w1start0, n1 = win_bounds(f1lo, f1hi)
    ca = rank_win(0, wstart_of(w0start0, 0), win_a, sem_a)
    cb = rank_win(1, wstart_of(w1start0, 0), win_b, sem_b)
    zero_buf(chunk_a)
    zero_buf(chunk_b)

    ca.wait()
    p1_scan(win_a, chunk_a, wstart_of(w0start0, 0), f0lo, f0hi)

    def more0(w, carry):
        wst = wstart_of(w0start0, w)
        rank_win(0, wst, win_a, sem_a).wait()
        p1_scan(win_a, chunk_a, wst, f0lo, f0hi)
        return carry
    lax.fori_loop(1, n0, more0, 0)
    wr0 = pltpu.async_copy(chunk_a,
                           lists_hbm.at[c, 0, pl.ds(orow, 16), :], sem_a)

    cb.wait()
    p1_scan(win_b, chunk_b, wstart_of(w1start0, 0), f1lo, f1hi)

    def more1(w, carry):
        wst = wstart_of(w1start0, w)
        rank_win(1, wst, win_b, sem_b).wait()
        p1_scan(win_b, chunk_b, wst, f1lo, f1hi)
        return carry
    lax.fori_loop(1, n1, more1, 0)
    wr1 = pltpu.async_copy(chunk_b,
                           lists_hbm.at[c, 1, pl.ds(orow, 16), :], sem_b)
    wr0.wait()
    wr1.wait()
    plsc.subcore_barrier()

    cp_bits.wait()
    zero_buf(chunk_a)

    jlo = extract(64, s)
    jhi = jnp.minimum(M, extract(64, s + 1) + 272)
    wstart0, nwin = win_bounds(jlo, jhi)

    def wloop2(w, carry):
        wstart = pl.multiple_of(jnp.minimum(wstart0 + w * WIN, M - WIN),
                                2048)
        wr = pl.multiple_of(wstart >> 8, 8)
        ca = pltpu.async_copy(lists_hbm.at[c, 0, pl.ds(wr, WROW), :],
                              win_a, sem_a)
        cb = pltpu.async_copy(lists_hbm.at[c, 1, pl.ds(wr, WROW), :],
                              win_b, sem_b)
        ca.wait()
        cb.wait()
        glo, ghi = grp_bounds(jlo, jhi, wstart)

        def g(i4, carry2):
            for u in range(4):
                i = i4 * 4 + u
                l0 = win_a[i >> 4, pl.ds((i & 15) * 16, 16)]
                l1 = win_b[i >> 4, pl.ds((i & 15) * 16, 16)]
                hx = l0 >> 8
                hy = l0 & 255
                tgt = ((hx - rt_fix(hx)) << 8) + hy - rt_fix(hy)
                w1 = l1 >> 4
                wv = plsc.load_gather(bitsb, [w1 & 15, w1 >> 4])
                v = (wv >> (l1 & 15)) & 1
                loc = tgt - lo_out
                m = (v == 1) & (loc >= 0) & (loc < CHUNK)
                lc = jnp.where(m, loc, 0)
                plsc.store_scatter(chunk_a, [lc >> 8, lc & 255], ones16,
                                   mask=m)
            return carry2
        lax.fori_loop(glo >> 2, (ghi + 3) >> 2, g, 0)
        return carry
    lax.fori_loop(0, nwin, wloop2, 0)

    count0 = extract(64, 16)
    count1 = extract(96, 0)

    @pl.when(jnp.logical_and(s == 0, count0 < M))
    def _():
        te = jnp.minimum(M, jnp.maximum(count0, count1) + 1)
        tw0, tnwin = win_bounds(count0, te)

        def wloop3(w, carry):
            wstart = pl.multiple_of(jnp.minimum(tw0 + w * WIN, M - WIN),
                                    2048)
            wr = pl.multiple_of(wstart >> 8, 8)
            pltpu.sync_copy(lists_hbm.at[c, 1, pl.ds(wr, WROW), :], win_b)
            glo, ghi = grp_bounds(count0, te, wstart)

            def g(i, carry2):
                j = wstart + i * 16 + lane
                l1 = win_b[i >> 4, pl.ds((i & 15) * 16, 16)]
                w1 = l1 >> 4
                wv = plsc.load_gather(bitsb, [w1 & 15, w1 >> 4])
                v = (wv >> (l1 & 15)) & 1
                m = (j >= count0) & (v == 1)
                plsc.store_scatter(chunk_a, [zeros16, zeros16], ones16,
                                   mask=m)
                return carry2
            lax.fori_loop(glo, ghi, g, 0)
            return carry
        lax.fori_loop(0, tnwin, wloop3, 0)

    pltpu.sync_copy(chunk_a, out_hbm.at[c, pl.ds(orow, CHUNK // 256), :])


@jax.jit
def _k2(rankpack, bounds, bits):
    mesh = plsc.VectorSubcoreMesh(core_axis_name="c", subcore_axis_name="s")
    return pl.kernel(
        _k2_body,
        out_type=[
            jax.ShapeDtypeStruct((2, H, H), jnp.int32),
            jax.ShapeDtypeStruct((2, 2, H, H), jnp.int32),
        ],
        mesh=mesh,
        compiler_params=pltpu.CompilerParams(needs_layout_passes=False,
                                             use_tc_tiling_on_sc=True),
        scratch_types=[
            pltpu.VMEM((NS, H), jnp.int32),
            pltpu.VMEM((WROW, H), jnp.int32),
            pltpu.VMEM((WROW, H), jnp.int32),
            pltpu.VMEM((CHUNK // 256, H), jnp.int32),
            pltpu.VMEM((CHUNK // 256, H), jnp.int32),
            pltpu.VMEM((2, 128), jnp.int32),
            pltpu.SemaphoreType.DMA,
            pltpu.SemaphoreType.DMA,
            pltpu.SemaphoreType.DMA,
        ],
    )(rankpack, bounds, bits)


def _k3_body(grids_ref, iou_ref):
    o = grids_ref[0] > 0
    p = grids_ref[1] > 0
    inter = jnp.sum((o & p).astype(jnp.float32))
    union = jnp.sum((o | p).astype(jnp.float32))
    iou_ref[0, 0] = inter / union


@jax.jit
def _k3(grids):
    return pl.pallas_call(
        _k3_body,
        in_specs=[pl.BlockSpec(memory_space=pltpu.VMEM)],
        out_specs=pl.BlockSpec(memory_space=pltpu.SMEM),
        out_shape=jax.ShapeDtypeStruct((1, 1), jnp.float32),
    )(grids)


def kernel(added_points, original_points, boxes, ego_loc):
    rankpack, bounds, bits = _k1(added_points, original_points,
                                 boxes, ego_loc)
    grids, _ = _k2(rankpack, bounds, bits)
    iou = _k3(grids)
    return iou[0, 0]


# speedup vs baseline: 1.2872x; 1.0026x over previous
import numpy as np

import jax
import jax.numpy as jnp
from jax import lax
from jax.experimental import pallas as pl
from jax.experimental.pallas import tpu as pltpu
from jax.experimental.pallas import tpu_sc as plsc

H = 256
M = H * H
T = 50

NS = 16
CHUNK = M // NS
GRP = CHUNK // 16

_rt = ((np.arange(H, dtype=np.float32) - np.float32(128.0))
       * np.float32(0.8))
_rt = (_rt / np.float32(0.8) + np.float32(128.0)).astype(np.int32)
_RMAP_NP = (_rt[:, None] * H + _rt[None, :]).astype(np.int32)


def _k1_body(added_ref, orig_ref, boxes_ref, ego_ref,
             rankpack_ref, bounds_ref, bits_ref):
    f32 = jnp.float32
    ii = lax.broadcasted_iota(jnp.int32, (H, H), 0).astype(f32)
    jj = lax.broadcasted_iota(jnp.int32, (H, H), 1).astype(f32)
    su = (ii < jj).astype(f32)
    sl = (ii > jj).astype(f32)
    ones = jnp.ones((H, H), f32)

    def ranks(nzf):
        row_tot = jnp.dot(nzf, ones, preferred_element_type=f32)
        row_prefix = jnp.dot(sl, row_tot, preferred_element_type=f32)
        row_pre = jnp.dot(nzf, su, preferred_element_type=f32)
        return row_prefix + row_pre

    for c, b in ((0, 0), (0, 1), (1, 0), (1, 1)):
        if c == 0:
            s = (orig_ref[b, 1] + orig_ref[b, 2] + orig_ref[b, 3]
                 + orig_ref[b, 4] + orig_ref[b, 5])
        else:
            s = (added_ref[b, 0] + added_ref[b, 1] + added_ref[b, 2]
                 + added_ref[b, 3] + added_ref[b, 4])
        nz = (s != 0.0)
        nzf = nz.astype(f32)
        rank = ranks(nzf).astype(jnp.int32)
        rankpack_ref[c, b] = rank | (nz.astype(jnp.int32) << 30)

        role = b
        off = role * 32
        bounds_ref[c, off] = 0
        for sb in range(1, NS):
            cnt = jnp.sum((rank < sb * CHUNK).astype(jnp.int32))
            bounds_ref[c, off + sb] = cnt
        bounds_ref[c, off + NS] = M
        if role == 0:
            nzi = nz.astype(jnp.int32)
            bounds_ref[c, 64] = 0
            for sb in range(1, NS):
                cnt = jnp.sum(nzi * (ii < float(sb * NS)).astype(jnp.int32))
                bounds_ref[c, 64 + sb] = cnt
            bounds_ref[c, 64 + NS] = jnp.sum(nzi)
        else:
            bounds_ref[c, 96] = jnp.sum(nz.astype(jnp.int32))

    px_col = (lax.broadcasted_iota(jnp.int32, (H, 1), 0).astype(f32)
              - 128.0) * 0.8
    py_row = (lax.broadcasted_iota(jnp.int32, (1, H), 1).astype(f32)
              - 128.0) * 0.8
    ego_x = ego_ref[1, 0]
    ego_y = ego_ref[1, 1]
    anyin = jnp.zeros((H, H), jnp.int32)
    for t in range(T):
        cx = boxes_ref[1, t, 0] - ego_x
        cy = boxes_ref[1, t, 1] - ego_y
        cz = boxes_ref[1, t, 2]
        dx = boxes_ref[1, t, 3]
        dy = boxes_ref[1, t, 4]
        dz = boxes_ref[1, t, 5]
        ry = boxes_ref[1, t, 6]
        cth = jnp.cos(-ry)
        sth = jnp.sin(-ry)
        ax = px_col - cx
        by = py_row - cy
        axc = ax * cth
        axs = ax * sth
        byc = by * cth
        bys = by * sth
        lx = axc - bys
        ly = axs + byc
        pz = 0.0 - cz
        zok = jnp.logical_and(pz >= 0.0, pz <= dz)
        inb = (jnp.abs(lx) <= dx * 0.5) & (jnp.abs(ly) <= dy * 0.5) & zok
        anyin = anyin | inb.astype(jnp.int32)

    ww = lax.broadcasted_iota(jnp.int32, (NS, H), 0)
    cc = lax.broadcasted_iota(jnp.int32, (NS, H), 1)
    pf = jnp.where((cc >> 4) == ww, 1 << (cc & 15), 0).astype(f32)
    bits_ref[...] = lax.dot_general(
        pf, anyin.astype(f32), (((1,), (1,)), ((), ())),
        preferred_element_type=f32).astype(jnp.int32)


@jax.jit
def _k1(added_points, original_points, boxes, ego_loc):
    return pl.pallas_call(
        _k1_body,
        in_specs=[
            pl.BlockSpec(memory_space=pltpu.VMEM),
            pl.BlockSpec(memory_space=pltpu.VMEM),
            pl.BlockSpec(memory_space=pltpu.SMEM),
            pl.BlockSpec(memory_space=pltpu.SMEM),
        ],
        out_specs=[
            pl.BlockSpec(memory_space=pltpu.VMEM),
            pl.BlockSpec(memory_space=pltpu.SMEM),
            pl.BlockSpec(memory_space=pltpu.VMEM),
        ],
        out_shape=[
            jax.ShapeDtypeStruct((2, 2, H, H), jnp.int32),
            jax.ShapeDtypeStruct((2, 128), jnp.int32),
            jax.ShapeDtypeStruct((NS, H), jnp.int32),
        ],
    )(added_points, original_points, boxes, ego_loc)


WIN = 8192
WROW = WIN // 256


def _k2_body(rankpack_hbm, bounds_hbm, bits_hbm, out_hbm, lists_hbm,
             bitsb, win_a, win_b, chunk_a, chunk_b, bvec,
             sem_a, sem_b, sem_bits):
    c = lax.axis_index("c")
    s = lax.axis_index("s")
    lo_out = s * CHUNK
    orow = pl.multiple_of(s * (CHUNK // 256), 8)
    lane = lax.iota(jnp.int32, 16)
    ones16 = jnp.ones((16,), jnp.int32)
    zeros16 = jnp.zeros((16,), jnp.int32)

    pltpu.sync_copy(bounds_hbm, bvec)
    cp_bits = pltpu.async_copy(bits_hbm, bitsb, sem_bits)

    def rt_fix(k):
        q = (k * 52429) >> 18
        return ((k - q * 5 == 4) & (k < 45)).astype(jnp.int32)

    def extract(off, k):
        va = bvec[c, pl.ds(off, 16)]
        vb = bvec[c, pl.ds(off + 16, 16)]
        return (jnp.sum(jnp.where(lane == k, va, 0))
                + jnp.sum(jnp.where(lane == (k - 16), vb, 0)))

    def zero_buf(buf):
        def z(i, carry):
            buf[i >> 4, pl.ds((i & 15) * 16, 16)] = zeros16
            return carry
        lax.fori_loop(0, GRP, z, 0, unroll=16)

    def win_bounds(vlo, vhi):
        wstart0 = (vlo >> 11) * 2048
        wend = ((vhi + 2047) >> 11) * 2048
        nwin = (wend - wstart0 + WIN - 1) >> 13
        return wstart0, nwin

    def grp_bounds(vlo, vhi, wstart):
        glo = jnp.maximum(0, (vlo - wstart) >> 4)
        ghi = jnp.minimum(WIN // 16,
                          jnp.maximum(glo, (vhi - wstart + 15) >> 4))
        return glo, ghi

    def wstart_of(ws0, w):
        return pl.multiple_of(jnp.minimum(ws0 + w * WIN, M - WIN), 2048)

    def rank_win(role, wstart, buf, sem):
        wr = pl.multiple_of(wstart >> 8, 8)
        return pltpu.async_copy(
            rankpack_hbm.at[c, role, pl.ds(wr, WROW), :], buf, sem)

    def p1_scan(buf, chunk, wstart, flo, fhi):
        glo, ghi = grp_bounds(flo, fhi, wstart)

        def g(i4, carry2):
            for u in range(4):
                i = i4 * 4 + u
                rp = buf[i >> 4, pl.ds((i & 15) * 16, 16)]
                rank = rp & 0xFFFF
                nz = rp >> 30
                f = wstart + i * 16 + lane
                loc = rank - lo_out
                m = (nz == 1) & (loc >= 0) & (loc < CHUNK)
                lc = jnp.where(m, loc, 0)
                plsc.store_scatter(chunk, [lc >> 8, lc & 255], f, mask=m)
            return carry2
        lax.fori_loop(glo >> 2, (ghi + 3) >> 2, g, 0)

    f0lo = extract(0, s)
    f0hi = extract(0, s + 1)
    f1lo = extract(32, s)
    f1hi = extract(32, s + 1)
    w0start0, n0 = win_bounds(f0lo, f0hi)
    w1start0, n1 = win_bounds(f1lo, f1hi)
    ca = rank_win(0, wstart_of(w0start0, 0), win_a, sem_a)
    cb = rank_win(1, wstart_of(w1start0, 0), win_b, sem_b)
    zero_buf(chunk_a)
    zero_buf(chunk_b)

    ca.wait()
    p1_scan(win_a, chunk_a, wstart_of(w0start0, 0), f0lo, f0hi)

    def more0(w, carry):
        wst = wstart_of(w0start0, w)
        rank_win(0, wst, win_a, sem_a).wait()
        p1_scan(win_a, chunk_a, wst, f0lo, f0hi)
        return carry
    lax.fori_loop(1, n0, more0, 0)
    wr0 = pltpu.async_copy(chunk_a,
                           lists_hbm.at[c, 0, pl.ds(orow, 16), :], sem_a)

    cb.wait()
    p1_scan(win_b, chunk_b, wstart_of(w1start0, 0), f1lo, f1hi)

    def more1(w, carry):
        wst = wstart_of(w1start0, w)
        rank_win(1, wst, win_b, sem_b).wait()
        p1_scan(win_b, chunk_b, wst, f1lo, f1hi)
        return carry
    lax.fori_loop(1, n1, more1, 0)
    wr1 = pltpu.async_copy(chunk_b,
                           lists_hbm.at[c, 1, pl.ds(orow, 16), :], sem_b)
    wr0.wait()
    wr1.wait()
    plsc.subcore_barrier()

    cp_bits.wait()
    zero_buf(chunk_a)

    jlo = extract(64, s)
    jhi = jnp.minimum(M, extract(64, s + 1) + 272)
    wstart0, nwin = win_bounds(jlo, jhi)

    def wloop2(w, carry):
        wstart = pl.multiple_of(jnp.minimum(wstart0 + w * WIN, M - WIN),
                                2048)
        wr = pl.multiple_of(wstart >> 8, 8)
        ca = pltpu.async_copy(lists_hbm.at[c, 0, pl.ds(wr, WROW), :],
                              win_a, sem_a)
        cb = pltpu.async_copy(lists_hbm.at[c, 1, pl.ds(wr, WROW), :],
                              win_b, sem_b)
        ca.wait()
        cb.wait()
        glo, ghi = grp_bounds(jlo, jhi, wstart)

        def g(i4, carry2):
            for u in range(4):
                i = i4 * 4 + u
                l0 = win_a[i >> 4, pl.ds((i & 15) * 16, 16)]
                l1 = win_b[i >> 4, pl.ds((i & 15) * 16, 16)]
                hx = l0 >> 8
                hy = l0 & 255
                tgt = ((hx - rt_fix(hx)) << 8) + hy - rt_fix(hy)
                w1 = l1 >> 4
                wv = plsc.load_gather(bitsb, [w1 & 15, w1 >> 4])
                v = (wv >> (l1 & 15)) & 1
                loc = tgt - lo_out
                m = (v == 1) & (loc >= 0) & (loc < CHUNK)
                lc = jnp.where(m, loc, 0)
                plsc.store_scatter(chunk_a, [lc >> 8, lc & 255], ones16,
                                   mask=m)
            return carry2
        lax.fori_loop(glo >> 2, (ghi + 3) >> 2, g, 0)
        return carry
    lax.fori_loop(0, nwin, wloop2, 0)

    count0 = extract(64, 16)
    count1 = extract(96, 0)

    @pl.when(jnp.logical_and(s == 0, count0 < M))
    def _():
        te = jnp.minimum(M, jnp.maximum(count0, count1) + 1)
        tw0, tnwin = win_bounds(count0, te)

        def wloop3(w, carry):
            wstart = pl.multiple_of(jnp.minimum(tw0 + w * WIN, M - WIN),
                                    2048)
            wr = pl.multiple_of(wstart >> 8, 8)
            pltpu.sync_copy(lists_hbm.at[c, 1, pl.ds(wr, WROW), :], win_b)
            glo, ghi = grp_bounds(count0, te, wstart)

            def g(i, carry2):
                j = wstart + i * 16 + lane
                l1 = win_b[i >> 4, pl.ds((i & 15) * 16, 16)]
                w1 = l1 >> 4
                wv = plsc.load_gather(bitsb, [w1 & 15, w1 >> 4])
                v = (wv >> (l1 & 15)) & 1
                m = (j >= count0) & (v == 1)
                plsc.store_scatter(chunk_a, [zeros16, zeros16], ones16,
                                   mask=m)
                return carry2
            lax.fori_loop(glo, ghi, g, 0)
            return carry
        lax.fori_loop(0, tnwin, wloop3, 0)

    pltpu.sync_copy(chunk_a, out_hbm.at[c, pl.ds(orow, CHUNK // 256), :])


@jax.jit
def _k2(rankpack, bounds, bits):
    mesh = plsc.VectorSubcoreMesh(core_axis_name="c", subcore_axis_name="s")
    return pl.kernel(
        _k2_body,
        out_type=[
            jax.ShapeDtypeStruct((2, H, H), jnp.int32),
            jax.ShapeDtypeStruct((2, 2, H, H), jnp.int32),
        ],
        mesh=mesh,
        compiler_params=pltpu.CompilerParams(needs_layout_passes=False,
                                             use_tc_tiling_on_sc=True),
        scratch_types=[
            pltpu.VMEM((NS, H), jnp.int32),
            pltpu.VMEM((WROW, H), jnp.int32),
            pltpu.VMEM((WROW, H), jnp.int32),
            pltpu.VMEM((CHUNK // 256, H), jnp.int32),
            pltpu.VMEM((CHUNK // 256, H), jnp.int32),
            pltpu.VMEM((2, 128), jnp.int32),
            pltpu.SemaphoreType.DMA,
            pltpu.SemaphoreType.DMA,
            pltpu.SemaphoreType.DMA,
        ],
    )(rankpack, bounds, bits)


def _k3_body(grids_ref, iou_ref):
    o = grids_ref[0] > 0
    p = grids_ref[1] > 0
    inter = jnp.sum((o & p).astype(jnp.float32))
    union = jnp.sum((o | p).astype(jnp.float32))
    iou_ref[0, 0] = inter / union


@jax.jit
def _k3(grids):
    return pl.pallas_call(
        _k3_body,
        in_specs=[pl.BlockSpec(memory_space=pltpu.VMEM)],
        out_specs=pl.BlockSpec(memory_space=pltpu.SMEM),
        out_shape=jax.ShapeDtypeStruct((1, 1), jnp.float32),
    )(grids)


def kernel(added_points, original_points, boxes, ego_loc):
    rankpack, bounds, bits = _k1(added_points, original_points,
                                 boxes, ego_loc)
    grids, _ = _k2(rankpack, bounds, bits)
    iou = _k3(grids)
    return iou[0, 0]
